# baseline (speedup 1.0000x reference)
import functools

import jax
import jax.numpy as jnp
from jax import lax
from jax.experimental import pallas as pl
from jax.experimental.pallas import tpu as pltpu
from jax.experimental.pallas import tpu_sc as plsc

D = 256
R = 4
HALF = 128
NC = 2
NS = 16
L = 16
K = 128
NB = 2
NP = 10112
RT = NP // NS


def _mm1_body(x_ref, w_ref, o_ref):
    o_ref[...] = jnp.dot(x_ref[...], w_ref[...],
                         preferred_element_type=jnp.float32)


def _mm2_body(a_ref, w_ref, o_ref):
    h0 = jnp.maximum(a_ref[0], 0.0)
    h1 = jnp.maximum(a_ref[1], 0.0)
    o_ref[...] = (
        jnp.dot(h0, w_ref[pl.ds(0, HALF), :], preferred_element_type=jnp.float32)
        + jnp.dot(h1, w_ref[pl.ds(HALF, HALF), :], preferred_element_type=jnp.float32))


def _mm1(x, wt, bn=1000):
    n = x.shape[0]
    return pl.pallas_call(
        _mm1_body,
        grid=(n // bn,),
        in_specs=[pl.BlockSpec((bn, D), lambda i: (i, 0)),
                  pl.BlockSpec((D, NC * R * HALF), lambda i: (0, 0))],
        out_specs=pl.BlockSpec((bn, NC * R * HALF), lambda i: (i, 0)),
        out_shape=jax.ShapeDtypeStruct((n, NC * R * HALF), jnp.float32),
    )(x, wt)


def _mm2(agg, wt, n, bn=1000):
    return pl.pallas_call(
        _mm2_body,
        grid=(n // bn,),
        in_specs=[pl.BlockSpec((NC, bn, HALF), lambda i: (0, i, 0)),
                  pl.BlockSpec((D, NC * R * HALF), lambda i: (0, 0))],
        out_specs=pl.BlockSpec((bn, NC * R * HALF), lambda i: (i, 0)),
        out_shape=jax.ShapeDtypeStruct((n, NC * R * HALF), jnp.float32),
    )(agg, wt)


def _sc_aggregate(table, cidxall, dstp, binit, tp, ch):
    mesh = plsc.VectorSubcoreMesh(core_axis_name="c", subcore_axis_name="s")
    rounds = ch // NB
    ep = NS * tp

    @functools.partial(
        pl.kernel, mesh=mesh,
        out_type=jax.ShapeDtypeStruct((NC * NP, HALF), jnp.float32),
        scratch_types=[
            pltpu.VMEM_SHARED((NP, HALF), jnp.float32),
            pltpu.VMEM((NB, K), jnp.int32),
            pltpu.VMEM((NB, K), jnp.int32),
            pltpu.VMEM((NB, K, HALF), jnp.float32),
            pltpu.SemaphoreType.DMA((NB,)),
            pltpu.SemaphoreType.DMA((NB,)),
            pltpu.SemaphoreType.DMA((NB,)),
        ],
    )
    def k(table_h, cidx_h, dst_h, binit_h, out_h,
          acc, cidx_v, didx_v, rows_v, isem, gsem, ssem):
        c = lax.axis_index("c")
        s = lax.axis_index("s")
        r0 = s * RT
        e0 = s * tp
        pltpu.sync_copy(binit_h.at[pl.ds(c * NP + r0, RT)], acc.at[pl.ds(r0, RT)])
        plsc.subcore_barrier()

        def fire_idx(j, b):
            pltpu.async_copy(cidx_h.at[pl.ds(c * ep + e0 + j * K, K)],
                             cidx_v.at[b], isem.at[b])
            pltpu.async_copy(dst_h.at[pl.ds(e0 + j * K, K)],
                             didx_v.at[b], isem.at[b])

        def wait_idx(b):
            pltpu.make_async_copy(cidx_h.at[pl.ds(e0, K)], cidx_v.at[b],
                                  isem.at[b]).wait()
            pltpu.make_async_copy(dst_h.at[pl.ds(e0, K)], didx_v.at[b],
                                  isem.at[b]).wait()

        def fire_gather(b):
            pltpu.async_copy(table_h.at[cidx_v.at[b]], rows_v.at[b], gsem.at[b])

        def wait_gather(b):
            pltpu.make_async_copy(table_h.at[cidx_v.at[b]], rows_v.at[b],
                                  gsem.at[b]).wait()

        def fire_scatter(b):
            pltpu.async_copy(rows_v.at[b], acc.at[didx_v.at[b]], ssem.at[b],
                             add=True)

        def wait_scatter(b):
            pltpu.make_async_copy(rows_v.at[b], acc.at[didx_v.at[b]],
                                  ssem.at[b]).wait()

        for b in range(NB):
            fire_idx(b, b)
        for b in range(NB):
            wait_idx(b)
            fire_gather(b)

        def round_body(jj, carry):
            for b in range(NB):
                wait_gather(b)
                fire_scatter(b)
            for b in range(NB):
                wait_scatter(b)
                fire_idx(jj * NB + b, b)
            for b in range(NB):
                wait_idx(b)
                fire_gather(b)
            return carry

        lax.fori_loop(1, rounds, round_body, 0)
        for b in range(NB):
            wait_gather(b)
            fire_scatter(b)
        for b in range(NB):
            wait_scatter(b)
        plsc.subcore_barrier()
        pltpu.sync_copy(acc.at[pl.ds(r0, RT)], out_h.at[pl.ds(c * NP + r0, RT)])

    return k(table, cidxall, dstp, binit)


def kernel(x, edge_index, edge_type, W1, b1, W2, b2):
    n = x.shape[0]
    e = edge_index.shape[1]
    src = edge_index[0]
    dst = edge_index[1]

    ch = -(-e // (NS * K))
    ch = -(-ch // NB) * NB
    tp = ch * K
    padn = NS * tp - e
    srcp = jnp.concatenate([src, jnp.zeros((padn,), jnp.int32)])
    dstp = jnp.concatenate([dst, jnp.full((padn,), n, jnp.int32)])
    relp = jnp.concatenate([edge_type, jnp.zeros((padn,), jnp.int32)])
    base_idx = srcp * (R * NC) + relp * NC
    cidxall = jnp.concatenate([base_idx, base_idx + 1])

    w1t = jnp.transpose(W1, (2, 0, 1)).reshape(D, R * D)
    w2t = jnp.transpose(W2, (2, 0, 1)).reshape(D, R * D)
    binit1 = jnp.broadcast_to(b1.reshape(NC, 1, HALF), (NC, NP, HALF)).reshape(NC * NP, HALF)
    binit2 = jnp.broadcast_to(b2.reshape(NC, 1, HALF), (NC, NP, HALF)).reshape(NC * NP, HALF)

    t1 = _mm1(x, w1t).reshape(n * R * NC, HALF)
    agg1 = _sc_aggregate(t1, cidxall, dstp, binit1, tp, ch)
    t2 = _mm2(agg1.reshape(NC, NP, HALF), w2t, n).reshape(n * R * NC, HALF)
    agg2 = _sc_aggregate(t2, cidxall, dstp, binit2, tp, ch)
    a2 = agg2.reshape(NC, NP, HALF)
    return jnp.concatenate([a2[0, :n], a2[1, :n]], axis=1)


# speedup vs baseline: 1.4781x; 1.4781x over previous
import functools

import jax
import jax.numpy as jnp
from jax import lax
from jax.experimental import pallas as pl
from jax.experimental.pallas import tpu as pltpu
from jax.experimental.pallas import tpu_sc as plsc

D = 256
R = 4
HALF = 128
NC = 2
NS = 16
L = 16
K = 112
NB = 2
NP = 10112
RT = NP // NS


def _mm1_body(x_ref, w_ref, o_ref):
    o_ref[...] = jnp.dot(x_ref[...], w_ref[...],
                         preferred_element_type=jnp.float32)


def _mm2_body(a_ref, w_ref, o_ref):
    h0 = jnp.maximum(a_ref[0], 0.0)
    h1 = jnp.maximum(a_ref[1], 0.0)
    o_ref[...] = (
        jnp.dot(h0, w_ref[pl.ds(0, HALF), :], preferred_element_type=jnp.float32)
        + jnp.dot(h1, w_ref[pl.ds(HALF, HALF), :], preferred_element_type=jnp.float32))


def _mm1(x, wt, bn=1000):
    n = x.shape[0]
    return pl.pallas_call(
        _mm1_body,
        grid=(n // bn,),
        in_specs=[pl.BlockSpec((bn, D), lambda i: (i, 0)),
                  pl.BlockSpec((D, NC * R * HALF), lambda i: (0, 0))],
        out_specs=pl.BlockSpec((bn, NC * R * HALF), lambda i: (i, 0)),
        out_shape=jax.ShapeDtypeStruct((n, NC * R * HALF), jnp.float32),
    )(x, wt)


def _mm2(agg, wt, n, bn=1000):
    return pl.pallas_call(
        _mm2_body,
        grid=(n // bn,),
        in_specs=[pl.BlockSpec((NC, bn, HALF), lambda i: (0, i, 0)),
                  pl.BlockSpec((D, NC * R * HALF), lambda i: (0, 0))],
        out_specs=pl.BlockSpec((bn, NC * R * HALF), lambda i: (i, 0)),
        out_shape=jax.ShapeDtypeStruct((n, NC * R * HALF), jnp.float32),
    )(agg, wt)


def _sc_aggregate(table, cidxall, dstp, binit, tp, ch):
    mesh = plsc.VectorSubcoreMesh(core_axis_name="c", subcore_axis_name="s")
    rounds = ch // NB
    ep = NS * tp

    @functools.partial(
        pl.kernel, mesh=mesh,
        out_type=jax.ShapeDtypeStruct((NC * NP, HALF), jnp.float32),
        scratch_types=[
            pltpu.VMEM_SHARED((NP, HALF), jnp.float32),
            pltpu.VMEM((tp,), jnp.int32),
            pltpu.VMEM((tp,), jnp.int32),
            pltpu.VMEM((NB, K), jnp.int32),
            pltpu.VMEM((NB, K, HALF), jnp.float32),
            pltpu.SemaphoreType.DMA((NB,)),
            pltpu.SemaphoreType.DMA((NB,)),
        ],
    )
    def k(table_h, cidx_h, dst_h, binit_h, out_h,
          acc, cidx_v, dst_v, didx_v, rows_v, gsem, ssem):
        c = lax.axis_index("c")
        s = lax.axis_index("s")
        r0 = s * RT
        e0 = s * tp
        pltpu.sync_copy(binit_h.at[pl.ds(c * NP + r0, RT)], acc.at[pl.ds(r0, RT)])
        pltpu.sync_copy(cidx_h.at[pl.ds(c * ep + e0, tp)], cidx_v)
        pltpu.sync_copy(dst_h.at[pl.ds(e0, tp)], dst_v)
        plsc.subcore_barrier()

        def fire_gather(j, b):
            for i in range(K // L):
                didx_v[b, pl.ds(i * L, L)] = dst_v[pl.ds(j * K + i * L, L)]
            pltpu.async_copy(table_h.at[cidx_v.at[pl.ds(j * K, K)]],
                             rows_v.at[b], gsem.at[b])

        def wait_gather(b):
            pltpu.make_async_copy(table_h.at[cidx_v.at[pl.ds(0, K)]],
                                  rows_v.at[b], gsem.at[b]).wait()

        def fire_scatter(b):
            pltpu.async_copy(rows_v.at[b], acc.at[didx_v.at[b]], ssem.at[b],
                             add=True)

        def wait_scatter(b):
            pltpu.make_async_copy(rows_v.at[b], acc.at[didx_v.at[b]],
                                  ssem.at[b]).wait()

        for b in range(NB):
            fire_gather(b, b)

        def round_body(jj, carry):
            for b in range(NB):
                wait_gather(b)
                fire_scatter(b)
            for b in range(NB):
                wait_scatter(b)
                fire_gather(jj * NB + b, b)
            return carry

        lax.fori_loop(1, rounds, round_body, 0)
        for b in range(NB):
            wait_gather(b)
            fire_scatter(b)
        for b in range(NB):
            wait_scatter(b)
        plsc.subcore_barrier()
        pltpu.sync_copy(acc.at[pl.ds(r0, RT)], out_h.at[pl.ds(c * NP + r0, RT)])

    return k(table, cidxall, dstp, binit)


def kernel(x, edge_index, edge_type, W1, b1, W2, b2):
    n = x.shape[0]
    e = edge_index.shape[1]
    src = edge_index[0]
    dst = edge_index[1]

    ch = -(-e // (NS * K))
    ch = -(-ch // NB) * NB
    tp = ch * K
    padn = NS * tp - e
    srcp = jnp.concatenate([src, jnp.zeros((padn,), jnp.int32)])
    dstp = jnp.concatenate([dst, jnp.full((padn,), n, jnp.int32)])
    relp = jnp.concatenate([edge_type, jnp.zeros((padn,), jnp.int32)])
    base_idx = srcp * (R * NC) + relp * NC
    cidxall = jnp.concatenate([base_idx, base_idx + 1])

    w1t = jnp.transpose(W1, (2, 0, 1)).reshape(D, R * D)
    w2t = jnp.transpose(W2, (2, 0, 1)).reshape(D, R * D)
    binit1 = jnp.broadcast_to(b1.reshape(NC, 1, HALF), (NC, NP, HALF)).reshape(NC * NP, HALF)
    binit2 = jnp.broadcast_to(b2.reshape(NC, 1, HALF), (NC, NP, HALF)).reshape(NC * NP, HALF)

    t1 = _mm1(x, w1t).reshape(n * R * NC, HALF)
    agg1 = _sc_aggregate(t1, cidxall, dstp, binit1, tp, ch)
    t2 = _mm2(agg1.reshape(NC, NP, HALF), w2t, n).reshape(n * R * NC, HALF)
    agg2 = _sc_aggregate(t2, cidxall, dstp, binit2, tp, ch)
    a2 = agg2.reshape(NC, NP, HALF)
    return jnp.concatenate([a2[0, :n], a2[1, :n]], axis=1)


# speedup vs baseline: 1.6959x; 1.1473x over previous
import functools

import jax
import jax.numpy as jnp
from jax import lax
from jax.experimental import pallas as pl
from jax.experimental.pallas import tpu as pltpu
from jax.experimental.pallas import tpu_sc as plsc

D = 256
R = 4
HALF = 128
NC = 2
NS = 16
L = 16
K = 120
NB = 3
NI = 2 * NB
NP = 10112
RT = NP // NS


def _mm1_body(x_ref, w_ref, o_ref):
    o_ref[...] = jnp.dot(x_ref[...], w_ref[...],
                         preferred_element_type=jnp.float32)


def _mm2_body(a_ref, w_ref, o_ref):
    h = jnp.maximum(a_ref[...], 0.0)
    o_ref[...] = jnp.dot(h, w_ref[...], preferred_element_type=jnp.float32)


def _mm1(x, wt, bn=1000):
    n = x.shape[0]
    return pl.pallas_call(
        _mm1_body,
        grid=(n // bn,),
        in_specs=[pl.BlockSpec((bn, D), lambda i: (i, 0)),
                  pl.BlockSpec((D, NC * R * HALF), lambda i: (0, 0))],
        out_specs=pl.BlockSpec((bn, NC * R * HALF), lambda i: (i, 0)),
        out_shape=jax.ShapeDtypeStruct((n, NC * R * HALF), jnp.float32),
    )(x, wt)


def _mm2(agg, wt, n, bn=1000):
    return pl.pallas_call(
        _mm2_body,
        grid=(n // bn,),
        in_specs=[pl.BlockSpec((bn, D), lambda i: (i, 0)),
                  pl.BlockSpec((D, NC * R * HALF), lambda i: (0, 0))],
        out_specs=pl.BlockSpec((bn, NC * R * HALF), lambda i: (i, 0)),
        out_shape=jax.ShapeDtypeStruct((n, NC * R * HALF), jnp.float32),
    )(agg, wt)


def _sc_aggregate(table, cidxall, dstp, binit, tp, ch):
    mesh = plsc.VectorSubcoreMesh(core_axis_name="c", subcore_axis_name="s")
    rounds = ch // NB
    ep = NS * tp

    @functools.partial(
        pl.kernel, mesh=mesh,
        out_type=jax.ShapeDtypeStruct((NP, NC * HALF), jnp.float32),
        scratch_types=[
            pltpu.VMEM_SHARED((NP, HALF), jnp.float32),
            pltpu.VMEM((NI, K), jnp.int32),
            pltpu.VMEM((NI, K), jnp.int32),
            pltpu.VMEM((NB, K, HALF), jnp.float32),
            pltpu.SemaphoreType.DMA((NI,)),
            pltpu.SemaphoreType.DMA((NB,)),
            pltpu.SemaphoreType.DMA((NB,)),
        ],
    )
    def k(table_h, cidx_h, dst_h, binit_h, out_h,
          acc, cidx_v, didx_v, rows_v, isem, gsem, ssem):
        c = lax.axis_index("c")
        s = lax.axis_index("s")
        r0 = s * RT
        e0 = s * tp
        pltpu.sync_copy(binit_h.at[pl.ds(c * NP + r0, RT)], acc.at[pl.ds(r0, RT)])
        plsc.subcore_barrier()

        def fire_idx(j, i):
            pltpu.async_copy(cidx_h.at[pl.ds(c * ep + e0 + j * K, K)],
                             cidx_v.at[i], isem.at[i])
            pltpu.async_copy(dst_h.at[pl.ds(e0 + j * K, K)],
                             didx_v.at[i], isem.at[i])

        def wait_idx(i):
            pltpu.make_async_copy(cidx_h.at[pl.ds(e0, K)], cidx_v.at[i],
                                  isem.at[i]).wait()
            pltpu.make_async_copy(dst_h.at[pl.ds(e0, K)], didx_v.at[i],
                                  isem.at[i]).wait()

        def fire_gather(i, b):
            pltpu.async_copy(table_h.at[cidx_v.at[i]], rows_v.at[b], gsem.at[b])

        def wait_gather(b):
            pltpu.make_async_copy(table_h.at[cidx_v.at[0]], rows_v.at[b],
                                  gsem.at[b]).wait()

        def fire_scatter(i, b):
            pltpu.async_copy(rows_v.at[b], acc.at[didx_v.at[i]], ssem.at[b],
                             add=True)

        def wait_scatter(i, b):
            pltpu.make_async_copy(rows_v.at[b], acc.at[didx_v.at[i]],
                                  ssem.at[b]).wait()

        for j in range(2 * NB):
            fire_idx(j, j)
        for b in range(NB):
            wait_idx(b)
            fire_gather(b, b)

        def round_body(jj, carry):
            par = ((jj - 1) % 2) * NB
            cur = (jj % 2) * NB
            for b in range(NB):
                wait_gather(b)
                fire_scatter(par + b, b)
            for b in range(NB):
                wait_scatter(par + b, b)
                wait_idx(cur + b)
                fire_gather(cur + b, b)
            for b in range(NB):
                fire_idx(jnp.minimum((jj + 1) * NB + b, ch - 1), par + b)
            return carry

        lax.fori_loop(1, rounds, round_body, 0)
        parf = ((rounds - 1) % 2) * NB
        for b in range(NB):
            wait_gather(b)
            fire_scatter(parf + b, b)
        for b in range(NB):
            wait_scatter(parf + b, b)
        for b in range(NB):
            wait_idx((rounds % 2) * NB + b)
        plsc.subcore_barrier()
        pltpu.sync_copy(acc.at[pl.ds(r0, RT)],
                        out_h.at[pl.ds(r0, RT),
                                 pl.ds(pl.multiple_of(c * HALF, HALF), HALF)])

    return k(table, cidxall, dstp, binit)


def kernel(x, edge_index, edge_type, W1, b1, W2, b2):
    n = x.shape[0]
    e = edge_index.shape[1]
    src = edge_index[0]
    dst = edge_index[1]

    ch = -(-e // (NS * K))
    ch = -(-ch // NB) * NB
    tp = ch * K
    padn = NS * tp - e
    srcp = jnp.concatenate([src, jnp.zeros((padn,), jnp.int32)])
    dstp = jnp.concatenate([dst, jnp.full((padn,), n, jnp.int32)])
    relp = jnp.concatenate([edge_type, jnp.zeros((padn,), jnp.int32)])
    base_idx = srcp * (R * NC) + relp * NC
    cidxall = jnp.concatenate([base_idx, base_idx + 1])

    w1t = jnp.transpose(W1, (2, 0, 1)).reshape(D, R * D)
    w2t = jnp.transpose(W2, (2, 0, 1)).reshape(D, R * D)
    binit1 = jnp.broadcast_to(b1.reshape(NC, 1, HALF), (NC, NP, HALF)).reshape(NC * NP, HALF)
    binit2 = jnp.broadcast_to(b2.reshape(NC, 1, HALF), (NC, NP, HALF)).reshape(NC * NP, HALF)

    t1 = _mm1(x, w1t).reshape(n * R * NC, HALF)
    agg1 = _sc_aggregate(t1, cidxall, dstp, binit1, tp, ch)
    t2 = _mm2(agg1, w2t, n).reshape(n * R * NC, HALF)
    agg2 = _sc_aggregate(t2, cidxall, dstp, binit2, tp, ch)
    return agg2[:n]


# speedup vs baseline: 1.7108x; 1.0088x over previous
import functools

import jax
import jax.numpy as jnp
from jax import lax
from jax.experimental import pallas as pl
from jax.experimental.pallas import tpu as pltpu
from jax.experimental.pallas import tpu_sc as plsc

D = 256
R = 4
HALF = 128
NC = 2
NS = 16
L = 16
K = 120
NB = 3
NI = 2 * NB
NP = 10112
RT = NP // NS


def _mm1_body(x_ref, w_ref, o_ref):
    o_ref[...] = jnp.dot(x_ref[...], w_ref[...],
                         preferred_element_type=jnp.float32)


def _mm2_body(a_ref, w_ref, o_ref):
    h = jnp.maximum(a_ref[...], 0.0)
    o_ref[...] = jnp.dot(h, w_ref[...], preferred_element_type=jnp.float32)


def _mm1(x, wt, bn=1000):
    n = x.shape[0]
    return pl.pallas_call(
        _mm1_body,
        grid=(n // bn,),
        in_specs=[pl.BlockSpec((bn, D), lambda i: (i, 0)),
                  pl.BlockSpec((D, NC * R * HALF), lambda i: (0, 0))],
        out_specs=pl.BlockSpec((bn, NC * R * HALF), lambda i: (i, 0)),
        out_shape=jax.ShapeDtypeStruct((n, NC * R * HALF), jnp.float32),
    )(x, wt)


def _mm2(agg, wt, n, bn=1000):
    return pl.pallas_call(
        _mm2_body,
        grid=(n // bn,),
        in_specs=[pl.BlockSpec((bn, D), lambda i: (i, 0)),
                  pl.BlockSpec((D, NC * R * HALF), lambda i: (0, 0))],
        out_specs=pl.BlockSpec((bn, NC * R * HALF), lambda i: (i, 0)),
        out_shape=jax.ShapeDtypeStruct((n, NC * R * HALF), jnp.float32),
    )(agg, wt)


def _sc_aggregate(table, cidxall, dstp, binit, tp, ch, n):
    mesh = plsc.VectorSubcoreMesh(core_axis_name="c", subcore_axis_name="s")
    rounds = ch // NB
    ep = NS * tp
    last_rows = n - (NS - 1) * RT

    @functools.partial(
        pl.kernel, mesh=mesh,
        out_type=jax.ShapeDtypeStruct((n, NC * HALF), jnp.float32),
        scratch_types=[
            pltpu.VMEM_SHARED((NP, HALF), jnp.float32),
            pltpu.VMEM((NI, K), jnp.int32),
            pltpu.VMEM((NI, K), jnp.int32),
            pltpu.VMEM((NB, K, HALF), jnp.float32),
            pltpu.SemaphoreType.DMA((NI,)),
            pltpu.SemaphoreType.DMA((NB,)),
            pltpu.SemaphoreType.DMA((NB,)),
        ],
    )
    def k(table_h, cidx_h, dst_h, binit_h, out_h,
          acc, cidx_v, didx_v, rows_v, isem, gsem, ssem):
        c = lax.axis_index("c")
        s = lax.axis_index("s")
        r0 = s * RT
        e0 = s * tp
        pltpu.sync_copy(binit_h.at[pl.ds(c * NP + r0, RT)], acc.at[pl.ds(r0, RT)])
        plsc.subcore_barrier()

        def fire_idx(j, i):
            pltpu.async_copy(cidx_h.at[pl.ds(c * ep + e0 + j * K, K)],
                             cidx_v.at[i], isem.at[i])
            pltpu.async_copy(dst_h.at[pl.ds(e0 + j * K, K)],
                             didx_v.at[i], isem.at[i])

        def wait_idx(i):
            pltpu.make_async_copy(cidx_h.at[pl.ds(e0, K)], cidx_v.at[i],
                                  isem.at[i]).wait()
            pltpu.make_async_copy(dst_h.at[pl.ds(e0, K)], didx_v.at[i],
                                  isem.at[i]).wait()

        def fire_gather(i, b):
            pltpu.async_copy(table_h.at[cidx_v.at[i]], rows_v.at[b], gsem.at[b])

        def wait_gather(b):
            pltpu.make_async_copy(table_h.at[cidx_v.at[0]], rows_v.at[b],
                                  gsem.at[b]).wait()

        def fire_scatter(i, b):
            pltpu.async_copy(rows_v.at[b], acc.at[didx_v.at[i]], ssem.at[b],
                             add=True)

        def wait_scatter(i, b):
            pltpu.make_async_copy(rows_v.at[b], acc.at[didx_v.at[i]],
                                  ssem.at[b]).wait()

        for j in range(2 * NB):
            fire_idx(j, j)
        for b in range(NB):
            wait_idx(b)
            fire_gather(b, b)

        def round_body(jj, carry):
            par = ((jj - 1) % 2) * NB
            cur = (jj % 2) * NB
            for b in range(NB):
                wait_gather(b)
                fire_scatter(par + b, b)
            for b in range(NB):
                wait_scatter(par + b, b)
                wait_idx(cur + b)
                fire_gather(cur + b, b)
            for b in range(NB):
                fire_idx(jnp.minimum((jj + 1) * NB + b, ch - 1), par + b)
            return carry

        lax.fori_loop(1, rounds, round_body, 0)
        parf = ((rounds - 1) % 2) * NB
        for b in range(NB):
            wait_gather(b)
            fire_scatter(parf + b, b)
        for b in range(NB):
            wait_scatter(parf + b, b)
        for b in range(NB):
            wait_idx((rounds % 2) * NB + b)
        plsc.subcore_barrier()

        @pl.when(s < NS - 1)
        def _():
            pltpu.sync_copy(acc.at[pl.ds(r0, RT)],
                            out_h.at[pl.ds(r0, RT),
                                     pl.ds(pl.multiple_of(c * HALF, HALF), HALF)])

        @pl.when(s == NS - 1)
        def _():
            pltpu.sync_copy(acc.at[pl.ds(r0, last_rows)],
                            out_h.at[pl.ds(r0, last_rows),
                                     pl.ds(pl.multiple_of(c * HALF, HALF), HALF)])

    return k(table, cidxall, dstp, binit)


def kernel(x, edge_index, edge_type, W1, b1, W2, b2):
    n = x.shape[0]
    e = edge_index.shape[1]
    src = edge_index[0]
    dst = edge_index[1]

    ch = -(-e // (NS * K))
    ch = -(-ch // NB) * NB
    tp = ch * K
    padn = NS * tp - e
    srcp = jnp.concatenate([src, jnp.zeros((padn,), jnp.int32)])
    dstp = jnp.concatenate([dst, jnp.full((padn,), n, jnp.int32)])
    relp = jnp.concatenate([edge_type, jnp.zeros((padn,), jnp.int32)])
    base_idx = srcp * (R * NC) + relp * NC
    cidxall = jnp.concatenate([base_idx, base_idx + 1])

    w1t = jnp.transpose(W1, (2, 0, 1)).reshape(D, R * D)
    w2t = jnp.transpose(W2, (2, 0, 1)).reshape(D, R * D)
    binit1 = jnp.broadcast_to(b1.reshape(NC, 1, HALF), (NC, NP, HALF)).reshape(NC * NP, HALF)
    binit2 = jnp.broadcast_to(b2.reshape(NC, 1, HALF), (NC, NP, HALF)).reshape(NC * NP, HALF)

    t1 = _mm1(x, w1t).reshape(n * R * NC, HALF)
    agg1 = _sc_aggregate(t1, cidxall, dstp, binit1, tp, ch, n)
    t2 = _mm2(agg1, w2t, n).reshape(n * R * NC, HALF)
    agg2 = _sc_aggregate(t2, cidxall, dstp, binit2, tp, ch, n)
    return agg2


# speedup vs baseline: 1.7153x; 1.0026x over previous
import functools

import jax
import jax.numpy as jnp
from jax import lax
from jax.experimental import pallas as pl
from jax.experimental.pallas import tpu as pltpu
from jax.experimental.pallas import tpu_sc as plsc

D = 256
R = 4
HALF = 128
NC = 2
NS = 16
L = 16
K = 120
NB = 3
NI = 2 * NB
NP = 10112
RT = NP // NS


def _mm1_body(x_ref, w_ref, o_ref):
    o_ref[...] = jnp.dot(x_ref[...], w_ref[...],
                         preferred_element_type=jnp.float32)


def _mm2_body(a_ref, w_ref, o_ref):
    h = jnp.maximum(a_ref[...], 0.0)
    o_ref[...] = jnp.dot(h, w_ref[...], preferred_element_type=jnp.float32)


def _mm1(x, wt, bn=1000):
    n = x.shape[0]
    return pl.pallas_call(
        _mm1_body,
        grid=(n // bn,),
        in_specs=[pl.BlockSpec((bn, D), lambda i: (i, 0)),
                  pl.BlockSpec((D, NC * R * HALF), lambda i: (0, 0))],
        out_specs=pl.BlockSpec((bn, NC * R * HALF), lambda i: (i, 0)),
        out_shape=jax.ShapeDtypeStruct((n, NC * R * HALF), jnp.float32),
    )(x, wt)


def _mm2(agg, wt, n, bn=1000):
    return pl.pallas_call(
        _mm2_body,
        grid=(n // bn,),
        in_specs=[pl.BlockSpec((bn, D), lambda i: (i, 0)),
                  pl.BlockSpec((D, NC * R * HALF), lambda i: (0, 0))],
        out_specs=pl.BlockSpec((bn, NC * R * HALF), lambda i: (i, 0)),
        out_shape=jax.ShapeDtypeStruct((n, NC * R * HALF), jnp.float32),
    )(agg, wt)


def _sc_aggregate(table, cidxall, dstp, binit, tp, ch, n):
    mesh = plsc.VectorSubcoreMesh(core_axis_name="c", subcore_axis_name="s")
    rounds = ch // NB
    ep = NS * tp
    last_rows = n - (NS - 1) * RT

    @functools.partial(
        pl.kernel, mesh=mesh,
        out_type=jax.ShapeDtypeStruct((n, NC * HALF), jnp.float32),
        scratch_types=[
            pltpu.VMEM_SHARED((NP, HALF), jnp.float32),
            pltpu.VMEM((NI, K), jnp.int32),
            pltpu.VMEM((NI, K), jnp.int32),
            pltpu.VMEM((NB, K, HALF), jnp.float32),
            pltpu.SemaphoreType.DMA((NI,)),
            pltpu.SemaphoreType.DMA((NB,)),
            pltpu.SemaphoreType.DMA((NB,)),
        ],
    )
    def k(table_h, cidx_h, dst_h, binit_h, out_h,
          acc, cidx_v, didx_v, rows_v, isem, gsem, ssem):
        c = lax.axis_index("c")
        s = lax.axis_index("s")
        r0 = s * RT
        e0 = s * tp
        pltpu.sync_copy(binit_h.at[pl.ds(c * RT, RT)], acc.at[pl.ds(r0, RT)])
        plsc.subcore_barrier()

        def fire_idx(j, i):
            pltpu.async_copy(cidx_h.at[pl.ds(c * ep + e0 + j * K, K)],
                             cidx_v.at[i], isem.at[i])
            pltpu.async_copy(dst_h.at[pl.ds(e0 + j * K, K)],
                             didx_v.at[i], isem.at[i])

        def wait_idx(i):
            pltpu.make_async_copy(cidx_h.at[pl.ds(e0, K)], cidx_v.at[i],
                                  isem.at[i]).wait()
            pltpu.make_async_copy(dst_h.at[pl.ds(e0, K)], didx_v.at[i],
                                  isem.at[i]).wait()

        def fire_gather(i, b):
            pltpu.async_copy(table_h.at[cidx_v.at[i]], rows_v.at[b], gsem.at[b])

        def wait_gather(b):
            pltpu.make_async_copy(table_h.at[cidx_v.at[0]], rows_v.at[b],
                                  gsem.at[b]).wait()

        def fire_scatter(i, b):
            pltpu.async_copy(rows_v.at[b], acc.at[didx_v.at[i]], ssem.at[b],
                             add=True)

        def wait_scatter(i, b):
            pltpu.make_async_copy(rows_v.at[b], acc.at[didx_v.at[i]],
                                  ssem.at[b]).wait()

        for j in range(2 * NB):
            fire_idx(j, j)
        for b in range(NB):
            wait_idx(b)
            fire_gather(b, b)

        def round_body(jj, carry):
            par = ((jj - 1) % 2) * NB
            cur = (jj % 2) * NB
            for b in range(NB):
                wait_gather(b)
                fire_scatter(par + b, b)
            for b in range(NB):
                wait_scatter(par + b, b)
                wait_idx(cur + b)
                fire_gather(cur + b, b)
            for b in range(NB):
                fire_idx(jnp.minimum((jj + 1) * NB + b, ch - 1), par + b)
            return carry

        lax.fori_loop(1, rounds, round_body, 0)
        parf = ((rounds - 1) % 2) * NB
        for b in range(NB):
            wait_gather(b)
            fire_scatter(parf + b, b)
        for b in range(NB):
            wait_scatter(parf + b, b)
        for b in range(NB):
            wait_idx((rounds % 2) * NB + b)
        plsc.subcore_barrier()

        @pl.when(s < NS - 1)
        def _():
            pltpu.sync_copy(acc.at[pl.ds(r0, RT)],
                            out_h.at[pl.ds(r0, RT),
                                     pl.ds(pl.multiple_of(c * HALF, HALF), HALF)])

        @pl.when(s == NS - 1)
        def _():
            pltpu.sync_copy(acc.at[pl.ds(r0, last_rows)],
                            out_h.at[pl.ds(r0, last_rows),
                                     pl.ds(pl.multiple_of(c * HALF, HALF), HALF)])

    return k(table, cidxall, dstp, binit)


def kernel(x, edge_index, edge_type, W1, b1, W2, b2):
    n = x.shape[0]
    e = edge_index.shape[1]
    src = edge_index[0]
    dst = edge_index[1]

    ch = -(-e // (NS * K))
    ch = -(-ch // NB) * NB
    tp = ch * K
    padn = NS * tp - e
    srcp = jnp.concatenate([src, jnp.zeros((padn,), jnp.int32)])
    dstp = jnp.concatenate([dst, jnp.full((padn,), n, jnp.int32)])
    relp = jnp.concatenate([edge_type, jnp.zeros((padn,), jnp.int32)])
    base_idx = srcp * (R * NC) + relp * NC
    cidxall = jnp.concatenate([base_idx, base_idx + 1])

    w1t = jnp.transpose(W1, (2, 0, 1)).reshape(D, R * D)
    w2t = jnp.transpose(W2, (2, 0, 1)).reshape(D, R * D)
    binit1 = jnp.broadcast_to(b1.reshape(NC, 1, HALF), (NC, RT, HALF)).reshape(NC * RT, HALF)
    binit2 = jnp.broadcast_to(b2.reshape(NC, 1, HALF), (NC, RT, HALF)).reshape(NC * RT, HALF)

    t1 = _mm1(x, w1t).reshape(n * R * NC, HALF)
    agg1 = _sc_aggregate(t1, cidxall, dstp, binit1, tp, ch, n)
    t2 = _mm2(agg1, w2t, n).reshape(n * R * NC, HALF)
    agg2 = _sc_aggregate(t2, cidxall, dstp, binit2, tp, ch, n)
    return agg2
